# Initial kernel scaffold; baseline (speedup 1.0000x reference)
import jax
import jax.numpy as jnp
from jax.experimental import pallas as pl


def kernel(x, edge_index, W1, b1, Wx, bx, W2, b2, bn_gamma, bn_beta):
    raise NotImplementedError("write your pallas kernel here")


# speedup vs baseline: 5.1021x; 5.1021x over previous
import functools

import jax
import jax.numpy as jnp
from jax import lax
from jax.experimental import pallas as pl
from jax.experimental.pallas import tpu as pltpu
from jax.experimental.pallas import tpu_sc as plsc

_N = 10000
_D = 128
_H = _D // 2
_K = 128
_NP = 10240
_NC = 2
_NS = 16
_RPT = _NP // _NS
_QC = _RPT // _K


def _agg_body(src_hbm, dst_hbm, h_hbm, out_hbm, src_t, dst_t, rows, zbuf, acc, sem):
    c = lax.axis_index("c")
    s = lax.axis_index("s")
    ch = src_t.shape[0]

    pltpu.sync_copy(src_hbm.at[c, s], src_t)
    pltpu.sync_copy(dst_hbm.at[s], dst_t)

    def _z(i, carry):
        for j in range(_H // 16):
            zbuf[i, pl.ds(j * 16, 16)] = jnp.zeros((16,), jnp.float32)
        return carry

    lax.fori_loop(0, _K, _z, 0)
    for q in range(_QC):
        pltpu.sync_copy(zbuf, acc.at[pl.ds(s * _RPT + q * _K, _K)])
    plsc.subcore_barrier()

    def _body(j, carry):
        pltpu.async_copy(h_hbm.at[src_t.at[j]], rows, sem).wait()
        pltpu.sync_copy(rows, acc.at[dst_t.at[j]], add=True)
        return carry

    lax.fori_loop(0, ch, _body, 0)
    plsc.subcore_barrier()

    for q in range(_QC):
        row0 = s * _RPT + q * _K
        pltpu.sync_copy(acc.at[pl.ds(row0, _K)], rows)
        pltpu.sync_copy(rows, out_hbm.at[c, pl.ds(row0, _K)])


@functools.lru_cache(maxsize=None)
def _make_agg(ch):
    mesh = plsc.VectorSubcoreMesh(
        core_axis_name="c", subcore_axis_name="s",
        num_cores=_NC, num_subcores=_NS)
    return pl.kernel(
        _agg_body,
        out_type=jax.ShapeDtypeStruct((_NC, _NP, _H), jnp.float32),
        mesh=mesh,
        compiler_params=pltpu.CompilerParams(use_tc_tiling_on_sc=False),
        scratch_types=[
            pltpu.VMEM((ch, _K), jnp.int32),
            pltpu.VMEM((ch, _K), jnp.int32),
            pltpu.VMEM((_K, _H), jnp.float32),
            pltpu.VMEM((_K, _H), jnp.float32),
            pltpu.VMEM_SHARED((_NP, _H), jnp.float32),
            pltpu.SemaphoreType.DMA,
        ],
    )


def _mm_body(h_ref, p_ref, w_ref, s_ref, t_ref, o_ref, *, relu):
    x0 = h_ref[0] + p_ref[0]
    x1 = h_ref[1] + p_ref[1]
    y = jnp.concatenate([x0, x1], axis=1)
    y = jnp.dot(y, w_ref[...], preferred_element_type=jnp.float32)
    y = y * s_ref[...] + t_ref[...]
    if relu:
        y = jnp.maximum(y, 0.0)
    o_ref[0] = y[:, :_H]
    o_ref[1] = y[:, _H:]


def _mm(h, parts, wt, scale, shift, relu):
    blk = 512
    return pl.pallas_call(
        functools.partial(_mm_body, relu=relu),
        grid=(_NP // blk,),
        in_specs=[
            pl.BlockSpec((_NC, blk, _H), lambda i: (0, i, 0)),
            pl.BlockSpec((_NC, blk, _H), lambda i: (0, i, 0)),
            pl.BlockSpec((_D, _D), lambda i: (0, 0)),
            pl.BlockSpec((1, _D), lambda i: (0, 0)),
            pl.BlockSpec((1, _D), lambda i: (0, 0)),
        ],
        out_specs=pl.BlockSpec((_NC, blk, _H), lambda i: (0, i, 0)),
        out_shape=jax.ShapeDtypeStruct((_NC, _NP, _H), jnp.float32),
    )(h, parts, wt, scale, shift)


def kernel(x, edge_index, W1, b1, Wx, bx, W2, b2, bn_gamma, bn_beta):
    src = edge_index[0]
    dst = edge_index[1]
    e = src.shape[0]
    per_tile = e // _NS
    ch = -(-per_tile // _K)
    pad = ch * _K - per_tile
    src_t = src.reshape(_NS, per_tile)
    dst_t = dst.reshape(_NS, per_tile)
    src_t = jnp.pad(src_t, ((0, 0), (0, pad))).reshape(_NS, ch, _K)
    dst_t = jnp.pad(dst_t, ((0, 0), (0, pad)), constant_values=_N).reshape(
        _NS, ch, _K)
    src_t = jnp.stack([src_t, src_t + _NP])

    xp = jnp.pad(x, ((0, _NP - _N), (0, 0)))
    h = jnp.stack([xp[:, :_H], xp[:, _H:]])
    agg = _make_agg(ch)

    one = jnp.ones((1, _D), jnp.float32)
    bn_scale = (bn_gamma / jnp.sqrt(jnp.float32(1.0 + 1e-5))).reshape(1, _D)
    layers = [
        (W1.T, one, b1.reshape(1, _D), True),
        (Wx.T, one, bx.reshape(1, _D), True),
        (W2.T, bn_scale, (b2 * bn_scale[0] + bn_beta).reshape(1, _D), False),
    ]
    for wt, sc, sh, relu in layers:
        parts = agg(src_t, dst_t, h.reshape(_NC * _NP, _H))
        h = _mm(h, parts, wt, sc, sh, relu)
    return jnp.concatenate([h[0, :_N], h[1, :_N]], axis=1)


# speedup vs baseline: 6.2770x; 1.2303x over previous
import functools

import jax
import jax.numpy as jnp
from jax import lax
from jax.experimental import pallas as pl
from jax.experimental.pallas import tpu as pltpu
from jax.experimental.pallas import tpu_sc as plsc

_N = 10000
_D = 128
_H = _D // 2
_K = 128
_NP = 10240
_NC = 2
_NS = 16
_RPT = _NP // _NS
_QC = _RPT // _K


def _agg_body(src_hbm, dst_hbm, h_hbm, out_hbm, src_t, dst_t, rows0, rows1,
              zbuf, acc, sem0, sem1):
    c = lax.axis_index("c")
    s = lax.axis_index("s")
    ch = src_t.shape[0]
    rows = (rows0, rows1)
    sems = (sem0, sem1)

    pltpu.sync_copy(src_hbm.at[c, s], src_t)
    pltpu.sync_copy(dst_hbm.at[s], dst_t)

    def _z(i, carry):
        for j in range(_H // 16):
            zbuf[i, pl.ds(j * 16, 16)] = jnp.zeros((16,), jnp.float32)
        return carry

    lax.fori_loop(0, _K, _z, 0)
    for q in range(_QC):
        pltpu.sync_copy(zbuf, acc.at[pl.ds(s * _RPT + q * _K, _K)])
    plsc.subcore_barrier()

    for b in range(2):
        pltpu.async_copy(h_hbm.at[src_t.at[b]], rows[b], sems[b])

    def _body(i, carry):
        for b in range(2):
            j = i * 2 + b
            pltpu.make_async_copy(h_hbm.at[src_t.at[j]], rows[b], sems[b]).wait()
            pltpu.sync_copy(rows[b], acc.at[dst_t.at[j]], add=True)

            @pl.when(j + 2 < ch)
            def _():
                pltpu.async_copy(h_hbm.at[src_t.at[j + 2]], rows[b], sems[b])
        return carry

    lax.fori_loop(0, ch // 2, _body, 0)
    plsc.subcore_barrier()

    for q in range(_QC):
        row0 = s * _RPT + q * _K
        pltpu.sync_copy(acc.at[pl.ds(row0, _K)], rows0)
        pltpu.sync_copy(rows0, out_hbm.at[c, pl.ds(row0, _K)])


@functools.lru_cache(maxsize=None)
def _make_agg(ch):
    mesh = plsc.VectorSubcoreMesh(
        core_axis_name="c", subcore_axis_name="s",
        num_cores=_NC, num_subcores=_NS)
    return pl.kernel(
        _agg_body,
        out_type=jax.ShapeDtypeStruct((_NC, _NP, _H), jnp.float32),
        mesh=mesh,
        compiler_params=pltpu.CompilerParams(use_tc_tiling_on_sc=False),
        scratch_types=[
            pltpu.VMEM((ch, _K), jnp.int32),
            pltpu.VMEM((ch, _K), jnp.int32),
            pltpu.VMEM((_K, _H), jnp.float32),
            pltpu.VMEM((_K, _H), jnp.float32),
            pltpu.VMEM((_K, _H), jnp.float32),
            pltpu.VMEM_SHARED((_NP, _H), jnp.float32),
            pltpu.SemaphoreType.DMA,
            pltpu.SemaphoreType.DMA,
        ],
    )


def _mm_body(h_ref, p_ref, w_ref, s_ref, t_ref, o_ref, *, relu):
    x0 = h_ref[0] + p_ref[0]
    x1 = h_ref[1] + p_ref[1]
    y = jnp.concatenate([x0, x1], axis=1)
    y = jnp.dot(y, w_ref[...], preferred_element_type=jnp.float32)
    y = y * s_ref[...] + t_ref[...]
    if relu:
        y = jnp.maximum(y, 0.0)
    o_ref[0] = y[:, :_H]
    o_ref[1] = y[:, _H:]


def _mm(h, parts, wt, scale, shift, relu):
    blk = 512
    return pl.pallas_call(
        functools.partial(_mm_body, relu=relu),
        grid=(_NP // blk,),
        in_specs=[
            pl.BlockSpec((_NC, blk, _H), lambda i: (0, i, 0)),
            pl.BlockSpec((_NC, blk, _H), lambda i: (0, i, 0)),
            pl.BlockSpec((_D, _D), lambda i: (0, 0)),
            pl.BlockSpec((1, _D), lambda i: (0, 0)),
            pl.BlockSpec((1, _D), lambda i: (0, 0)),
        ],
        out_specs=pl.BlockSpec((_NC, blk, _H), lambda i: (0, i, 0)),
        out_shape=jax.ShapeDtypeStruct((_NC, _NP, _H), jnp.float32),
    )(h, parts, wt, scale, shift)


def kernel(x, edge_index, W1, b1, Wx, bx, W2, b2, bn_gamma, bn_beta):
    src = edge_index[0]
    dst = edge_index[1]
    e = src.shape[0]
    per_tile = e // _NS
    ch = -(-per_tile // _K)
    ch += ch % 2
    pad = ch * _K - per_tile
    src_t = src.reshape(_NS, per_tile)
    dst_t = dst.reshape(_NS, per_tile)
    src_t = jnp.pad(src_t, ((0, 0), (0, pad))).reshape(_NS, ch, _K)
    dst_t = jnp.pad(dst_t, ((0, 0), (0, pad)), constant_values=_N).reshape(
        _NS, ch, _K)
    src_t = jnp.stack([src_t, src_t + _NP])

    xp = jnp.pad(x, ((0, _NP - _N), (0, 0)))
    h = jnp.stack([xp[:, :_H], xp[:, _H:]])
    agg = _make_agg(ch)

    one = jnp.ones((1, _D), jnp.float32)
    bn_scale = (bn_gamma / jnp.sqrt(jnp.float32(1.0 + 1e-5))).reshape(1, _D)
    layers = [
        (W1.T, one, b1.reshape(1, _D), True),
        (Wx.T, one, bx.reshape(1, _D), True),
        (W2.T, bn_scale, (b2 * bn_scale[0] + bn_beta).reshape(1, _D), False),
    ]
    for wt, sc, sh, relu in layers:
        parts = agg(src_t, dst_t, h.reshape(_NC * _NP, _H))
        h = _mm(h, parts, wt, sc, sh, relu)
    return jnp.concatenate([h[0, :_N], h[1, :_N]], axis=1)
